# baseline (device time: 11187 ns/iter reference)
import jax
import jax.numpy as jnp
from jax import lax
from jax.experimental import pallas as pl
from jax.experimental.pallas import tpu as pltpu


def kernel(x):
    m, n = x.shape
    half = m // 2
    q = half // 2

    def body(x_ref, out_ref, comm_ref, y_send, y_recv, xz_send, xz_recv):
        my_x = lax.axis_index("x")
        my_y = lax.axis_index("y")
        my_z = lax.axis_index("z")
        h0 = lax.rem(my_x + my_z, 2) * half

        peer_y = (my_x, 1 - my_y, my_z)
        peer_x = (1 - my_x, my_y, my_z)
        peer_z = (my_x, my_y, 1 - my_z)

        barrier_sem = pltpu.get_barrier_semaphore()
        for nbr in (peer_y, peer_x, peer_z):
            pl.semaphore_signal(
                barrier_sem, inc=1,
                device_id=nbr, device_id_type=pl.DeviceIdType.MESH,
            )
        pl.semaphore_wait(barrier_sem, 3)

        rdma_y = pltpu.make_async_remote_copy(
            src_ref=x_ref.at[pl.ds(h0, half)],
            dst_ref=comm_ref,
            send_sem=y_send,
            recv_sem=y_recv,
            device_id=peer_y,
            device_id_type=pl.DeviceIdType.MESH,
        )
        rdma_y.start()
        rdma_y.wait()

        out_ref[pl.ds(h0, half), :] = x_ref[pl.ds(h0, half), :] + comm_ref[...]

        rdma_x = pltpu.make_async_remote_copy(
            src_ref=out_ref.at[pl.ds(h0, q)],
            dst_ref=out_ref.at[pl.ds(h0, q)],
            send_sem=xz_send.at[0],
            recv_sem=xz_recv.at[0],
            device_id=peer_x,
            device_id_type=pl.DeviceIdType.MESH,
        )
        rdma_z = pltpu.make_async_remote_copy(
            src_ref=out_ref.at[pl.ds(h0 + q, q)],
            dst_ref=out_ref.at[pl.ds(h0 + q, q)],
            send_sem=xz_send.at[1],
            recv_sem=xz_recv.at[1],
            device_id=peer_z,
            device_id_type=pl.DeviceIdType.MESH,
        )
        rdma_x.start()
        rdma_z.start()
        rdma_x.wait()
        rdma_z.wait()

    return pl.pallas_call(
        body,
        out_shape=jax.ShapeDtypeStruct((m, n), x.dtype),
        in_specs=[pl.BlockSpec(memory_space=pltpu.VMEM)],
        out_specs=pl.BlockSpec(memory_space=pltpu.VMEM),
        scratch_shapes=[
            pltpu.VMEM((half, n), x.dtype),
            pltpu.SemaphoreType.DMA,
            pltpu.SemaphoreType.DMA,
            pltpu.SemaphoreType.DMA((2,)),
            pltpu.SemaphoreType.DMA((2,)),
        ],
        compiler_params=pltpu.CompilerParams(collective_id=0),
    )(x)


# device time: 8464 ns/iter; 1.3217x vs baseline; 1.3217x over previous
import jax
import jax.numpy as jnp
from jax import lax
from jax.experimental import pallas as pl
from jax.experimental.pallas import tpu as pltpu

NCHUNKS = 4


def kernel(x):
    m, n = x.shape
    rows = m // NCHUNKS

    def body(x_ref, out_ref, comm_ref, send_sems, recv_sems):
        my_x = lax.axis_index("x")
        my_y = lax.axis_index("y")
        my_z = lax.axis_index("z")
        peer = (my_x, 1 - my_y, my_z)

        barrier_sem = pltpu.get_barrier_semaphore()
        pl.semaphore_signal(
            barrier_sem, inc=1,
            device_id=peer, device_id_type=pl.DeviceIdType.MESH,
        )
        pl.semaphore_wait(barrier_sem, 1)

        rdmas = []
        for c in range(NCHUNKS):
            r = pltpu.make_async_remote_copy(
                src_ref=x_ref.at[pl.ds(c * rows, rows)],
                dst_ref=comm_ref.at[pl.ds(c * rows, rows)],
                send_sem=send_sems.at[c],
                recv_sem=recv_sems.at[c],
                device_id=peer,
                device_id_type=pl.DeviceIdType.MESH,
            )
            r.start()
            rdmas.append(r)

        for c in range(NCHUNKS):
            rdmas[c].wait_recv()
            sl = pl.ds(c * rows, rows)
            out_ref[sl, :] = x_ref[sl, :] + comm_ref[sl, :]

        for c in range(NCHUNKS):
            rdmas[c].wait_send()

    return pl.pallas_call(
        body,
        out_shape=jax.ShapeDtypeStruct((m, n), x.dtype),
        in_specs=[pl.BlockSpec(memory_space=pltpu.VMEM)],
        out_specs=pl.BlockSpec(memory_space=pltpu.VMEM),
        scratch_shapes=[
            pltpu.VMEM((m, n), x.dtype),
            pltpu.SemaphoreType.DMA((NCHUNKS,)),
            pltpu.SemaphoreType.DMA((NCHUNKS,)),
        ],
        compiler_params=pltpu.CompilerParams(collective_id=0),
    )(x)
